# baseline (speedup 1.0000x reference)
import jax
import jax.numpy as jnp
from jax import lax
from jax.experimental import pallas as pl
from jax.experimental.pallas import tpu as pltpu
from jax.experimental.pallas import tpu_sc as plsc

L = 16
NW = 32
N = 100000
CHUNK = 3136
LAST = N - (NW - 1) * CHUNK
NSEG = 1024
SEG_PAD = 1040
SENT = 1032
MAX_Z = 89


def _sc_body(po_hbm, nt_hbm, seg_hbm, out_hbm, po_v, nt_v, seg_v, s_v, e_v,
             sem_po, sem_nt, sem_seg):
    c = lax.axis_index("c")
    s = lax.axis_index("s")
    wid = s * 2 + c
    base = wid * CHUNK
    is_last = wid == NW - 1
    nvec = jnp.where(is_last, LAST // L, CHUNK // L)

    pltpu.make_async_copy(po_hbm, po_v.at[pl.ds(0, MAX_Z)], sem_po).start()

    @pl.when(jnp.logical_not(is_last))
    def _():
        pltpu.make_async_copy(
            nt_hbm.at[pl.ds(base, CHUNK)], nt_v, sem_nt).start()
        pltpu.make_async_copy(
            seg_hbm.at[pl.ds(base, CHUNK)], seg_v.at[pl.ds(0, CHUNK)],
            sem_seg).start()

    @pl.when(is_last)
    def _():
        pltpu.make_async_copy(
            nt_hbm.at[pl.ds(base, LAST)], nt_v.at[pl.ds(0, LAST)],
            sem_nt).start()
        pltpu.make_async_copy(
            seg_hbm.at[pl.ds(base, LAST)], seg_v.at[pl.ds(0, LAST)],
            sem_seg).start()

    zeros = jnp.zeros((L,), jnp.float32)

    @plsc.parallel_loop(0, SEG_PAD // L, unroll=4)
    def _(j):
        s_v[pl.ds(j * L, L)] = zeros
        e_v[pl.ds(j * L, L)] = zeros

    pltpu.make_async_copy(po_hbm, po_v.at[pl.ds(0, MAX_Z)], sem_po).wait()

    @pl.when(jnp.logical_not(is_last))
    def _():
        pltpu.make_async_copy(
            nt_hbm.at[pl.ds(base, CHUNK)], nt_v, sem_nt).wait()
        pltpu.make_async_copy(
            seg_hbm.at[pl.ds(base, CHUNK)], seg_v.at[pl.ds(0, CHUNK)],
            sem_seg).wait()

    @pl.when(is_last)
    def _():
        pltpu.make_async_copy(
            nt_hbm.at[pl.ds(base, LAST)], nt_v.at[pl.ds(0, LAST)],
            sem_nt).wait()
        pltpu.make_async_copy(
            seg_hbm.at[pl.ds(base, LAST)], seg_v.at[pl.ds(0, LAST)],
            sem_seg).wait()

    seg_v[pl.ds(nvec * L, L)] = jnp.full((L,), SENT, jnp.int32)

    @plsc.parallel_loop(0, nvec, unroll=8, carry=jnp.float32(0.0))
    def _(i, carry):
        off = i * L
        seg = seg_v[pl.ds(off, L)]
        seg_next = seg_v[pl.ds(off + 1, L)]
        nt = nt_v[pl.ds(off, L)]
        v = plsc.load_gather(po_v, [nt])
        c_incl = plsc.cumsum(v) + carry
        m_end = seg != seg_next
        plsc.store_scatter(e_v, [seg], c_incl, mask=m_end)
        plsc.store_scatter(s_v, [seg_next], c_incl, mask=m_end)
        return lax.squeeze(lax.slice(c_incl, (L - 1,), (L,)), (0,))

    @plsc.parallel_loop(0, NSEG // L, unroll=4)
    def _(j):
        s_v[pl.ds(j * L, L)] = e_v[pl.ds(j * L, L)] - s_v[pl.ds(j * L, L)]

    pltpu.sync_copy(s_v.at[pl.ds(0, NSEG)], out_hbm.at[wid])


_sc_partials = pl.kernel(
    _sc_body,
    out_type=jax.ShapeDtypeStruct((NW, NSEG), jnp.float32),
    mesh=plsc.VectorSubcoreMesh(
        core_axis_name="c", subcore_axis_name="s", num_cores=2, num_subcores=16),
    compiler_params=pltpu.CompilerParams(
        needs_layout_passes=False, disable_bounds_checks=True, skip_device_barrier=True),
    scratch_types=[
        pltpu.VMEM((96,), jnp.float32),
        pltpu.VMEM((CHUNK,), jnp.int32),
        pltpu.VMEM((CHUNK + L,), jnp.int32),
        pltpu.VMEM((SEG_PAD,), jnp.float32),
        pltpu.VMEM((SEG_PAD,), jnp.float32),
        pltpu.SemaphoreType.DMA,
        pltpu.SemaphoreType.DMA,
        pltpu.SemaphoreType.DMA,
    ],
)


def _reduce_body(in_ref, out_ref):
    out_ref[...] = jnp.sum(in_ref[...], axis=0)


_tc_reduce = pl.pallas_call(
    _reduce_body,
    out_shape=jax.ShapeDtypeStruct((NSEG,), jnp.float32),
)


def kernel(property_offset, node_type, segment_ids):
    partials = _sc_partials(property_offset.astype(jnp.float32),
                            node_type.astype(jnp.int32),
                            segment_ids.astype(jnp.int32))
    return _tc_reduce(partials)


# speedup vs baseline: 1.0085x; 1.0085x over previous
import jax
import jax.numpy as jnp
from jax import lax
from jax.experimental import pallas as pl
from jax.experimental.pallas import tpu as pltpu
from jax.experimental.pallas import tpu_sc as plsc

L = 16
NW = 32
N = 100000
CHUNK = 3136
LAST = N - (NW - 1) * CHUNK
NSEG = 1024
SEG_PAD = 1040
SENT = 1032
MAX_Z = 89


def _sc_body(po_hbm, nt_hbm, seg_hbm, out_hbm, po_v, nt_v, seg_v, s_v, e_v,
             sem_po, sem_nt, sem_seg):
    c = lax.axis_index("c")
    s = lax.axis_index("s")
    wid = s * 2 + c
    base = wid * CHUNK
    is_last = wid == NW - 1
    nvec = jnp.where(is_last, LAST // L, CHUNK // L)

    pltpu.make_async_copy(po_hbm, po_v.at[pl.ds(0, MAX_Z)], sem_po).start()

    @pl.when(jnp.logical_not(is_last))
    def _():
        pltpu.make_async_copy(
            nt_hbm.at[pl.ds(base, CHUNK)], nt_v, sem_nt).start()
        pltpu.make_async_copy(
            seg_hbm.at[pl.ds(base, CHUNK)], seg_v.at[pl.ds(0, CHUNK)],
            sem_seg).start()

    @pl.when(is_last)
    def _():
        pltpu.make_async_copy(
            nt_hbm.at[pl.ds(base, LAST)], nt_v.at[pl.ds(0, LAST)],
            sem_nt).start()
        pltpu.make_async_copy(
            seg_hbm.at[pl.ds(base, LAST)], seg_v.at[pl.ds(0, LAST)],
            sem_seg).start()

    zeros = jnp.zeros((L,), jnp.float32)

    @plsc.parallel_loop(0, SEG_PAD // L, unroll=4)
    def _(j):
        s_v[pl.ds(j * L, L)] = zeros
        e_v[pl.ds(j * L, L)] = zeros

    pltpu.make_async_copy(po_hbm, po_v.at[pl.ds(0, MAX_Z)], sem_po).wait()

    @pl.when(jnp.logical_not(is_last))
    def _():
        pltpu.make_async_copy(
            nt_hbm.at[pl.ds(base, CHUNK)], nt_v, sem_nt).wait()
        pltpu.make_async_copy(
            seg_hbm.at[pl.ds(base, CHUNK)], seg_v.at[pl.ds(0, CHUNK)],
            sem_seg).wait()

    @pl.when(is_last)
    def _():
        pltpu.make_async_copy(
            nt_hbm.at[pl.ds(base, LAST)], nt_v.at[pl.ds(0, LAST)],
            sem_nt).wait()
        pltpu.make_async_copy(
            seg_hbm.at[pl.ds(base, LAST)], seg_v.at[pl.ds(0, LAST)],
            sem_seg).wait()

    seg_v[pl.ds(nvec * L, L)] = jnp.full((L,), SENT, jnp.int32)

    @plsc.parallel_loop(0, nvec, unroll=4, carry=jnp.float32(0.0))
    def _(i, carry):
        off = i * L
        seg = seg_v[pl.ds(off, L)]
        seg_next = seg_v[pl.ds(off + 1, L)]
        nt = nt_v[pl.ds(off, L)]
        v = plsc.load_gather(po_v, [nt])
        c_incl = plsc.cumsum(v) + carry
        m_end = seg != seg_next
        plsc.store_scatter(e_v, [seg], c_incl, mask=m_end)
        plsc.store_scatter(s_v, [seg_next], c_incl, mask=m_end)
        return lax.squeeze(lax.slice(c_incl, (L - 1,), (L,)), (0,))

    @plsc.parallel_loop(0, NSEG // L, unroll=4)
    def _(j):
        s_v[pl.ds(j * L, L)] = e_v[pl.ds(j * L, L)] - s_v[pl.ds(j * L, L)]

    pltpu.sync_copy(s_v.at[pl.ds(0, NSEG)], out_hbm.at[wid])


_sc_partials = pl.kernel(
    _sc_body,
    out_type=jax.ShapeDtypeStruct((NW, NSEG), jnp.float32),
    mesh=plsc.VectorSubcoreMesh(
        core_axis_name="c", subcore_axis_name="s", num_cores=2, num_subcores=16),
    compiler_params=pltpu.CompilerParams(
        needs_layout_passes=False, disable_bounds_checks=True),
    scratch_types=[
        pltpu.VMEM((96,), jnp.float32),
        pltpu.VMEM((CHUNK,), jnp.int32),
        pltpu.VMEM((CHUNK + L,), jnp.int32),
        pltpu.VMEM((SEG_PAD,), jnp.float32),
        pltpu.VMEM((SEG_PAD,), jnp.float32),
        pltpu.SemaphoreType.DMA,
        pltpu.SemaphoreType.DMA,
        pltpu.SemaphoreType.DMA,
    ],
)


def _reduce_body(in_ref, out_ref):
    out_ref[...] = jnp.sum(in_ref[...], axis=0)


_tc_reduce = pl.pallas_call(
    _reduce_body,
    out_shape=jax.ShapeDtypeStruct((NSEG,), jnp.float32),
)


def kernel(property_offset, node_type, segment_ids):
    partials = _sc_partials(property_offset.astype(jnp.float32),
                            node_type.astype(jnp.int32),
                            segment_ids.astype(jnp.int32))
    return _tc_reduce(partials)
